# Initial kernel scaffold; baseline (speedup 1.0000x reference)
import jax
import jax.numpy as jnp
from jax.experimental import pallas as pl


def kernel(x, patch_W, patch_b, cls, ln1_w, ln1_b, attn_in_w, attn_in_b, attn_out_w, attn_out_b, ln2_w, ln2_b, router_W, router_b, exp_W1, exp_b1, exp_W2, exp_b2, head_W, head_b):
    raise NotImplementedError("write your pallas kernel here")


# speedup vs baseline: 2.0292x; 2.0292x over previous
import functools

import jax
import jax.numpy as jnp
from jax.experimental import pallas as pl

D = 768
HEADS = 8
DH = D // HEADS
E = 8
DFF = 4 * D
HCHUNK = 768
NHC = DFF // HCHUNK
PATCH = 16

_DOT = functools.partial(
    jax.lax.dot_general,
    precision=jax.lax.Precision.HIGHEST,
    preferred_element_type=jnp.float32,
)


def _mm_nt(a, b):
    return _DOT(a, b, dimension_numbers=(((a.ndim - 1,), (b.ndim - 1,)), ((), ())))


def _layernorm(x, w, b):
    m = jnp.mean(x, axis=-1, keepdims=True)
    v = jnp.mean((x - m) ** 2, axis=-1, keepdims=True)
    return (x - m) / jnp.sqrt(v + 1e-5) * w + b


def _matmul_bias_body(x_ref, w_ref, b_ref, o_ref):
    o_ref[...] = _mm_nt(x_ref[...], w_ref[...]) + b_ref[...]


def _ln_matmul_bias_body(x_ref, lw_ref, lb_ref, w_ref, b_ref, o_ref):
    x2 = _layernorm(x_ref[...], lw_ref[...], lb_ref[...])
    o_ref[...] = _mm_nt(x2, w_ref[...]) + b_ref[...]


def _cls_block_body(xsc_ref, l1w_ref, l1b_ref, wq_ref, bq_ref, k_ref, v_ref,
                    wo_ref, bo_ref, l2w_ref, l2b_ref, rw_ref, rb_ref,
                    xs1_ref, xf_ref, mask_ref):
    nb = xsc_ref.shape[0]
    xsc = xsc_ref[...]
    qc = _mm_nt(_layernorm(xsc, l1w_ref[...], l1b_ref[...]), wq_ref[...]) + bq_ref[...]
    hm = (jax.lax.broadcasted_iota(jnp.int32, (D, HEADS), 0) // DH
          == jax.lax.broadcasted_iota(jnp.int32, (D, HEADS), 1)).astype(jnp.float32)
    hmt = (jax.lax.broadcasted_iota(jnp.int32, (HEADS, D), 1) // DH
           == jax.lax.broadcasted_iota(jnp.int32, (HEADS, D), 0)).astype(jnp.float32)
    scale = 1.0 / jnp.sqrt(jnp.float32(DH))
    rows = []
    for b in range(nb):
        kb = k_ref[b]
        vb = v_ref[b]
        prod = kb * qc[b:b + 1, :]
        scores = _DOT(prod, hm, (((1,), (0,)), ((), ()))) * scale
        mx = jnp.max(scores, axis=0, keepdims=True)
        ex = jnp.exp(scores - mx)
        sm = ex / jnp.sum(ex, axis=0, keepdims=True)
        attn_e = _DOT(sm, hmt, (((1,), (0,)), ((), ())))
        rows.append(jnp.sum(attn_e * vb, axis=0, keepdims=True))
    o = jnp.concatenate(rows, axis=0)
    xs1 = xsc + _mm_nt(o, wo_ref[...]) + bo_ref[...]
    xf = _layernorm(xs1, l2w_ref[...], l2b_ref[...])
    logits = _mm_nt(xf, rw_ref[...]) + rb_ref[...]
    lane = jax.lax.broadcasted_iota(jnp.int32, (nb, E), 1)
    cnt = jnp.zeros((nb, E), jnp.float32)
    for ep in range(E):
        le = logits[:, ep:ep + 1]
        gt = (le > logits).astype(jnp.float32)
        eq = jnp.logical_and(le == logits, ep < lane).astype(jnp.float32)
        cnt = cnt + gt + eq
    mask_ref[...] = (cnt < 2.0).astype(jnp.float32)
    xs1_ref[...] = xs1
    xf_ref[...] = xf


def _moe_body(xf_ref, m_ref, w1_ref, b1_ref, w2_ref, b2_ref, o_ref):
    e = pl.program_id(0)
    hc = pl.program_id(1)

    @pl.when(jnp.logical_and(e == 0, hc == 0))
    def _():
        o_ref[...] = jnp.zeros_like(o_ref)

    xf = xf_ref[...]
    h = _mm_nt(xf, w1_ref[0]) + b1_ref[0, 0]
    g = h * 0.5 * (1.0 + jax.lax.erf(h * (2.0 ** -0.5)))
    contrib = _DOT(g, w2_ref[0], (((1,), (1,)), ((), ())))
    onehot = (jax.lax.broadcasted_iota(jnp.int32, (E, 1), 0) == e).astype(jnp.float32)
    mcol = _DOT(m_ref[...], onehot, (((1,), (0,)), ((), ())))

    @pl.when(hc == 0)
    def _():
        o_ref[...] += mcol * b2_ref[0]

    o_ref[...] += mcol * contrib


def _head_body(xs1_ref, moe_ref, w_ref, b_ref, o_ref):
    y = xs1_ref[...] + moe_ref[...]
    o_ref[...] = _mm_nt(y, w_ref[...]) + b_ref[...]


def kernel(x, patch_W, patch_b, cls, ln1_w, ln1_b, attn_in_w, attn_in_b,
           attn_out_w, attn_out_b, ln2_w, ln2_b, router_W, router_b,
           exp_W1, exp_b1, exp_W2, exp_b2, head_W, head_b):
    B = x.shape[0]
    Hp = x.shape[2] // PATCH
    Wp = x.shape[3] // PATCH
    N = B * Hp * Wp
    S = Hp * Wp + 1
    T = B * S
    NC = head_W.shape[0]

    patches = (x.reshape(B, 3, Hp, PATCH, Wp, PATCH)
                .transpose(0, 2, 4, 1, 3, 5).reshape(N, 3 * PATCH * PATCH))
    w_patch = patch_W.reshape(D, 3 * PATCH * PATCH)
    mt = N // 4
    emb = pl.pallas_call(
        _matmul_bias_body,
        grid=(4,),
        in_specs=[
            pl.BlockSpec((mt, 3 * PATCH * PATCH), lambda i: (i, 0)),
            pl.BlockSpec(w_patch.shape, lambda i: (0, 0)),
            pl.BlockSpec((1, D), lambda i: (0, 0)),
        ],
        out_specs=pl.BlockSpec((mt, D), lambda i: (i, 0)),
        out_shape=jax.ShapeDtypeStruct((N, D), jnp.float32),
    )(patches, w_patch, patch_b.reshape(1, D))

    cls_row = cls.reshape(1, D)
    xs_tok = jnp.concatenate(
        [jnp.broadcast_to(cls_row[None], (B, 1, D)), emb.reshape(B, Hp * Wp, D)],
        axis=1)
    xs_flat = xs_tok.reshape(T, D)
    Tpad = 1600
    xs_pad = jnp.pad(xs_flat, ((0, Tpad - T), (0, 0)))
    w_kv = attn_in_w[D:]
    b_kv = attn_in_b[D:]
    mt2 = Tpad // 4
    kv = pl.pallas_call(
        _ln_matmul_bias_body,
        grid=(4,),
        in_specs=[
            pl.BlockSpec((mt2, D), lambda i: (i, 0)),
            pl.BlockSpec((1, D), lambda i: (0, 0)),
            pl.BlockSpec((1, D), lambda i: (0, 0)),
            pl.BlockSpec((2 * D, D), lambda i: (0, 0)),
            pl.BlockSpec((1, 2 * D), lambda i: (0, 0)),
        ],
        out_specs=pl.BlockSpec((mt2, 2 * D), lambda i: (i, 0)),
        out_shape=jax.ShapeDtypeStruct((Tpad, 2 * D), jnp.float32),
    )(xs_pad, ln1_w.reshape(1, D), ln1_b.reshape(1, D), w_kv, b_kv.reshape(1, 2 * D))
    kv = kv[:T]
    k = kv[:, :D].reshape(B, S, D)
    v = kv[:, D:].reshape(B, S, D)

    xs_cls = jnp.broadcast_to(cls_row, (B, D))
    xs1, xf, maskf = pl.pallas_call(
        _cls_block_body,
        out_shape=[
            jax.ShapeDtypeStruct((B, D), jnp.float32),
            jax.ShapeDtypeStruct((B, D), jnp.float32),
            jax.ShapeDtypeStruct((B, E), jnp.float32),
        ],
    )(xs_cls, ln1_w.reshape(1, D), ln1_b.reshape(1, D),
      attn_in_w[:D], attn_in_b[:D].reshape(1, D), k, v,
      attn_out_w, attn_out_b.reshape(1, D),
      ln2_w.reshape(1, D), ln2_b.reshape(1, D),
      router_W, router_b.reshape(1, E))

    b1r = exp_b1.reshape(E, NHC, 1, HCHUNK)
    b2r = exp_b2.reshape(E, 1, D)
    moe = pl.pallas_call(
        _moe_body,
        grid=(E, NHC),
        in_specs=[
            pl.BlockSpec((B, D), lambda e, h: (0, 0)),
            pl.BlockSpec((B, E), lambda e, h: (0, 0)),
            pl.BlockSpec((1, HCHUNK, D), lambda e, h: (e, h, 0)),
            pl.BlockSpec((1, 1, 1, HCHUNK), lambda e, h: (e, h, 0, 0)),
            pl.BlockSpec((1, D, HCHUNK), lambda e, h: (e, 0, h)),
            pl.BlockSpec((1, 1, D), lambda e, h: (e, 0, 0)),
        ],
        out_specs=pl.BlockSpec((B, D), lambda e, h: (0, 0)),
        out_shape=jax.ShapeDtypeStruct((B, D), jnp.float32),
    )(xf, maskf, exp_W1, b1r, exp_W2, b2r)

    out = pl.pallas_call(
        _head_body,
        out_shape=jax.ShapeDtypeStruct((B, NC), jnp.float32),
    )(xs1, moe, head_W, head_b.reshape(1, NC))
    return out


# speedup vs baseline: 2.8276x; 1.3935x over previous
import functools

import jax
import jax.numpy as jnp
from jax.experimental import pallas as pl

D = 768
HEADS = 8
DH = D // HEADS
E = 8
DFF = 4 * D
HCHUNK = 768
NHC = DFF // HCHUNK
PATCH = 16

_DOT = functools.partial(
    jax.lax.dot_general,
    precision=jax.lax.Precision.HIGHEST,
    preferred_element_type=jnp.float32,
)
_DOTF = functools.partial(
    jax.lax.dot_general,
    precision=jax.lax.Precision.DEFAULT,
    preferred_element_type=jnp.float32,
)


def _mm_nt(a, b):
    return _DOT(a, b, dimension_numbers=(((a.ndim - 1,), (b.ndim - 1,)), ((), ())))


def _layernorm(x, w, b):
    m = jnp.mean(x, axis=-1, keepdims=True)
    v = jnp.mean((x - m) ** 2, axis=-1, keepdims=True)
    return (x - m) / jnp.sqrt(v + 1e-5) * w + b


def _embed_kv_body(p_ref, wp_ref, bp_ref, lw_ref, lb_ref, w_ref, b_ref, o_ref):
    emb = _mm_nt(p_ref[...], wp_ref[...]) + bp_ref[...]
    x2 = _layernorm(emb, lw_ref[...], lb_ref[...])
    o_ref[...] = _mm_nt(x2, w_ref[...]) + b_ref[...]


def _cls_block_body(clsr_ref, l1w_ref, l1b_ref, wqkv_ref, bqkv_ref, kv_ref,
                    wo_ref, bo_ref, l2w_ref, l2b_ref, rw_ref, rb_ref,
                    xs1_ref, xf_ref, mask_ref):
    nb = kv_ref.shape[0]
    clsr = clsr_ref[...]
    qkv_c = _mm_nt(_layernorm(clsr, l1w_ref[...], l1b_ref[...]),
                   wqkv_ref[...]) + bqkv_ref[...]
    qc = qkv_c[:, :D]
    kc = qkv_c[:, D:2 * D]
    vc = qkv_c[:, 2 * D:]
    hm = (jax.lax.broadcasted_iota(jnp.int32, (D, HEADS), 0) // DH
          == jax.lax.broadcasted_iota(jnp.int32, (D, HEADS), 1)).astype(jnp.float32)
    hmt = (jax.lax.broadcasted_iota(jnp.int32, (HEADS, D), 1) // DH
           == jax.lax.broadcasted_iota(jnp.int32, (HEADS, D), 0)).astype(jnp.float32)
    scale = 1.0 / jnp.sqrt(jnp.float32(DH))
    s_cls = _DOT(kc * qc, hm, (((1,), (0,)), ((), ()))) * scale
    rows = []
    for b in range(nb):
        kb = kv_ref[b][:, :D]
        vb = kv_ref[b][:, D:]
        prod = kb * qc
        s_p = _DOT(prod, hm, (((1,), (0,)), ((), ()))) * scale
        scores = jnp.concatenate([s_cls, s_p], axis=0)
        mx = jnp.max(scores, axis=0, keepdims=True)
        ex = jnp.exp(scores - mx)
        sm = ex / jnp.sum(ex, axis=0, keepdims=True)
        a_cls = _DOT(sm[:1], hmt, (((1,), (0,)), ((), ())))
        a_p = _DOT(sm[1:], hmt, (((1,), (0,)), ((), ())))
        ob = a_cls * vc + jnp.sum(a_p * vb, axis=0, keepdims=True)
        rows.append(ob)
    o = jnp.concatenate(rows, axis=0)
    xs1 = clsr + _mm_nt(o, wo_ref[...]) + bo_ref[...]
    xf = _layernorm(xs1, l2w_ref[...], l2b_ref[...])
    logits = _mm_nt(xf, rw_ref[...]) + rb_ref[...]
    lane = jax.lax.broadcasted_iota(jnp.int32, (nb, E), 1)
    cnt = jnp.zeros((nb, E), jnp.float32)
    for ep in range(E):
        le = logits[:, ep:ep + 1]
        gt = (le > logits).astype(jnp.float32)
        eq = jnp.logical_and(le == logits, ep < lane).astype(jnp.float32)
        cnt = cnt + gt + eq
    mask_ref[...] = (cnt < 2.0).astype(jnp.float32)
    xs1_ref[...] = xs1
    xf_ref[...] = xf


def _moe_body(xf_ref, m_ref, w1_ref, b1_ref, w2_ref, b2_ref, o_ref):
    e = pl.program_id(0)
    hc = pl.program_id(1)

    @pl.when(jnp.logical_and(e == 0, hc == 0))
    def _():
        o_ref[...] = jnp.zeros_like(o_ref)

    xf = xf_ref[...]
    h = _DOTF(xf, w1_ref[0], (((1,), (1,)), ((), ()))) + b1_ref[0, 0]
    g = h * 0.5 * (1.0 + jax.lax.erf(h * (2.0 ** -0.5)))
    contrib = _DOTF(g, w2_ref[0], (((1,), (1,)), ((), ())))
    onehot = (jax.lax.broadcasted_iota(jnp.int32, (E, 1), 0) == e).astype(jnp.float32)
    mcol = _DOT(m_ref[...], onehot, (((1,), (0,)), ((), ())))

    @pl.when(hc == 0)
    def _():
        o_ref[...] += mcol * b2_ref[0]

    o_ref[...] += mcol * contrib


def _head_body(xs1_ref, moe_ref, w_ref, b_ref, o_ref):
    y = xs1_ref[...] + moe_ref[...]
    o_ref[...] = _mm_nt(y, w_ref[...]) + b_ref[...]


def kernel(x, patch_W, patch_b, cls, ln1_w, ln1_b, attn_in_w, attn_in_b,
           attn_out_w, attn_out_b, ln2_w, ln2_b, router_W, router_b,
           exp_W1, exp_b1, exp_W2, exp_b2, head_W, head_b):
    B = x.shape[0]
    Hp = x.shape[2] // PATCH
    Wp = x.shape[3] // PATCH
    NP = Hp * Wp
    N = B * NP
    NC = head_W.shape[0]

    patches = (x.reshape(B, 3, Hp, PATCH, Wp, PATCH)
                .transpose(0, 2, 4, 1, 3, 5).reshape(N, 3 * PATCH * PATCH))
    w_patch = patch_W.reshape(D, 3 * PATCH * PATCH)
    mt = N // 4
    kv_emb = pl.pallas_call(
        _embed_kv_body,
        grid=(4,),
        in_specs=[
            pl.BlockSpec((mt, 3 * PATCH * PATCH), lambda i: (i, 0)),
            pl.BlockSpec(w_patch.shape, lambda i: (0, 0)),
            pl.BlockSpec((1, D), lambda i: (0, 0)),
            pl.BlockSpec((1, D), lambda i: (0, 0)),
            pl.BlockSpec((1, D), lambda i: (0, 0)),
            pl.BlockSpec((2 * D, D), lambda i: (0, 0)),
            pl.BlockSpec((1, 2 * D), lambda i: (0, 0)),
        ],
        out_specs=pl.BlockSpec((mt, 2 * D), lambda i: (i, 0)),
        out_shape=jax.ShapeDtypeStruct((N, 2 * D), jnp.float32),
    )(patches, w_patch, patch_b.reshape(1, D), ln1_w.reshape(1, D),
      ln1_b.reshape(1, D), attn_in_w[D:], attn_in_b[D:].reshape(1, 2 * D))
    kv = kv_emb.reshape(B, NP, 2 * D)

    xs1, xf, maskf = pl.pallas_call(
        _cls_block_body,
        out_shape=[
            jax.ShapeDtypeStruct((B, D), jnp.float32),
            jax.ShapeDtypeStruct((B, D), jnp.float32),
            jax.ShapeDtypeStruct((B, E), jnp.float32),
        ],
    )(cls.reshape(1, D), ln1_w.reshape(1, D), ln1_b.reshape(1, D),
      attn_in_w, attn_in_b.reshape(1, 3 * D), kv,
      attn_out_w, attn_out_b.reshape(1, D),
      ln2_w.reshape(1, D), ln2_b.reshape(1, D),
      router_W, router_b.reshape(1, E))

    b1r = exp_b1.reshape(E, NHC, 1, HCHUNK)
    b2r = exp_b2.reshape(E, 1, D)
    moe = pl.pallas_call(
        _moe_body,
        grid=(E, NHC),
        in_specs=[
            pl.BlockSpec((B, D), lambda e, h: (0, 0)),
            pl.BlockSpec((B, E), lambda e, h: (0, 0)),
            pl.BlockSpec((1, HCHUNK, D), lambda e, h: (e, h, 0)),
            pl.BlockSpec((1, 1, 1, HCHUNK), lambda e, h: (e, h, 0, 0)),
            pl.BlockSpec((1, D, HCHUNK), lambda e, h: (e, 0, h)),
            pl.BlockSpec((1, 1, D), lambda e, h: (e, 0, 0)),
        ],
        out_specs=pl.BlockSpec((B, D), lambda e, h: (0, 0)),
        out_shape=jax.ShapeDtypeStruct((B, D), jnp.float32),
    )(xf, maskf, exp_W1, b1r, exp_W2, b2r)

    out = pl.pallas_call(
        _head_body,
        out_shape=jax.ShapeDtypeStruct((B, NC), jnp.float32),
    )(xs1, moe, head_W, head_b.reshape(1, NC))
    return out


# speedup vs baseline: 3.6066x; 1.2755x over previous
import functools

import jax
import jax.numpy as jnp
from jax import lax
from jax.experimental import pallas as pl
from jax.experimental.pallas import tpu as pltpu
from jax.experimental.pallas import tpu_sc as plsc

D = 768
HEADS = 8
DH = D // HEADS
E = 8
DFF = 4 * D
HCHUNK = 768
NHC = DFF // HCHUNK
PATCH = 16

_DOT = functools.partial(
    jax.lax.dot_general,
    precision=jax.lax.Precision.HIGHEST,
    preferred_element_type=jnp.float32,
)
_DOTF = functools.partial(
    jax.lax.dot_general,
    precision=jax.lax.Precision.DEFAULT,
    preferred_element_type=jnp.float32,
)


def _mm_nt(a, b):
    return _DOT(a, b, dimension_numbers=(((a.ndim - 1,), (b.ndim - 1,)), ((), ())))


def _layernorm(x, w, b):
    m = jnp.mean(x, axis=-1, keepdims=True)
    v = jnp.mean((x - m) ** 2, axis=-1, keepdims=True)
    return (x - m) / jnp.sqrt(v + 1e-5) * w + b


def _embed_kv_body(p_ref, wp_ref, bp_ref, lw_ref, lb_ref, w_ref, b_ref, o_ref):
    emb = _mm_nt(p_ref[...], wp_ref[...]) + bp_ref[...]
    x2 = _layernorm(emb, lw_ref[...], lb_ref[...])
    o_ref[...] = _mm_nt(x2, w_ref[...]) + b_ref[...]


def _cls_block_body(clsr_ref, l1w_ref, l1b_ref, wqkv_ref, bqkv_ref, kv_ref,
                    wo_ref, bo_ref, l2w_ref, l2b_ref, rw_ref, rb_ref,
                    xs1_ref, xf_ref, mask_ref):
    nb = kv_ref.shape[0]
    clsr = clsr_ref[...]
    qkv_c = _mm_nt(_layernorm(clsr, l1w_ref[...], l1b_ref[...]),
                   wqkv_ref[...]) + bqkv_ref[...]
    qc = qkv_c[:, :D]
    kc = qkv_c[:, D:2 * D]
    vc = qkv_c[:, 2 * D:]
    hm = (jax.lax.broadcasted_iota(jnp.int32, (D, HEADS), 0) // DH
          == jax.lax.broadcasted_iota(jnp.int32, (D, HEADS), 1)).astype(jnp.float32)
    hmt = (jax.lax.broadcasted_iota(jnp.int32, (HEADS, D), 1) // DH
           == jax.lax.broadcasted_iota(jnp.int32, (HEADS, D), 0)).astype(jnp.float32)
    scale = 1.0 / jnp.sqrt(jnp.float32(DH))
    s_cls = _DOT(kc * qc, hm, (((1,), (0,)), ((), ()))) * scale
    rows = []
    for b in range(nb):
        kb = kv_ref[b][:, :D]
        vb = kv_ref[b][:, D:]
        prod = kb * qc
        s_p = _DOT(prod, hm, (((1,), (0,)), ((), ()))) * scale
        scores = jnp.concatenate([s_cls, s_p], axis=0)
        mx = jnp.max(scores, axis=0, keepdims=True)
        ex = jnp.exp(scores - mx)
        sm = ex / jnp.sum(ex, axis=0, keepdims=True)
        a_cls = _DOT(sm[:1], hmt, (((1,), (0,)), ((), ())))
        a_p = _DOT(sm[1:], hmt, (((1,), (0,)), ((), ())))
        ob = a_cls * vc + jnp.sum(a_p * vb, axis=0, keepdims=True)
        rows.append(ob)
    o = jnp.concatenate(rows, axis=0)
    xs1 = clsr + _mm_nt(o, wo_ref[...]) + bo_ref[...]
    xf = _layernorm(xs1, l2w_ref[...], l2b_ref[...])
    logits = _mm_nt(xf, rw_ref[...]) + rb_ref[...]
    lane = jax.lax.broadcasted_iota(jnp.int32, (nb, E), 1)
    cnt = jnp.zeros((nb, E), jnp.float32)
    for ep in range(E):
        le = logits[:, ep:ep + 1]
        gt = (le > logits).astype(jnp.float32)
        eq = jnp.logical_and(le == logits, ep < lane).astype(jnp.float32)
        cnt = cnt + gt + eq
    mask_ref[...] = (cnt < 2.0).astype(jnp.float32)
    xs1_ref[...] = xs1
    xf_ref[...] = xf


def _moe_body(xf_ref, m_ref, w1_ref, b1_ref, w2_ref, b2_ref, o_ref):
    e = pl.program_id(0)
    hc = pl.program_id(1)

    @pl.when(jnp.logical_and(e == 0, hc == 0))
    def _():
        o_ref[...] = jnp.zeros_like(o_ref)

    xf = xf_ref[...]
    h = _DOTF(xf, w1_ref[0], (((1,), (1,)), ((), ()))) + b1_ref[0, 0]
    g = h * 0.5 * (1.0 + jax.lax.erf(h * (2.0 ** -0.5)))
    contrib = _DOTF(g, w2_ref[0], (((1,), (1,)), ((), ())))
    onehot = (jax.lax.broadcasted_iota(jnp.int32, (E, 1), 0) == e).astype(jnp.float32)
    mcol = _DOT(m_ref[...], onehot, (((1,), (0,)), ((), ())))

    @pl.when(hc == 0)
    def _():
        o_ref[...] += mcol * b2_ref[0]

    o_ref[...] += mcol * contrib


def _head_body(xs1_ref, moe_ref, w_ref, b_ref, o_ref):
    y = xs1_ref[...] + moe_ref[...]
    o_ref[...] = _mm_nt(y, w_ref[...]) + b_ref[...]


def kernel(x, patch_W, patch_b, cls, ln1_w, ln1_b, attn_in_w, attn_in_b,
           attn_out_w, attn_out_b, ln2_w, ln2_b, router_W, router_b,
           exp_W1, exp_b1, exp_W2, exp_b2, head_W, head_b):
    B = x.shape[0]
    Hp = x.shape[2] // PATCH
    Wp = x.shape[3] // PATCH
    NP = Hp * Wp
    N = B * NP
    NC = head_W.shape[0]

    NSEG = 3 * PATCH
    NROWS = N * NSEG
    NW = 32
    per_w = NROWS // NW
    NCHB = 112
    NCH = per_w // NCHB
    j = jnp.arange(NROWS, dtype=jnp.int32)
    t, s = j // NSEG, j % NSEG
    b_i, r_i = t // NP, t % NP
    hp_i, wp_i = r_i // Wp, r_i % Wp
    c_i, py_i = s // PATCH, s % PATCH
    src = ((b_i * 3 + c_i) * (Hp * PATCH) + hp_i * PATCH + py_i) * Wp + wp_i
    idx3 = src.reshape(NW, NCH, NCHB)
    x_rows = x.reshape(NROWS, PATCH)

    mesh = plsc.VectorSubcoreMesh(core_axis_name="c", subcore_axis_name="s")

    @functools.partial(
        pl.kernel,
        mesh=mesh,
        compiler_params=pltpu.CompilerParams(use_tc_tiling_on_sc=False),
        out_type=jax.ShapeDtypeStruct((NW, NCH, NCHB, PATCH), jnp.float32),
        scratch_types=[
            pltpu.VMEM((NCH, NCHB), jnp.int32),
            pltpu.VMEM((NCH, NCHB, PATCH), jnp.float32),
            pltpu.SemaphoreType.DMA,
        ],
    )
    def _patch_gather(x_hbm, idx_hbm, out_hbm, idx_v, rows_v, sem):
        w = lax.axis_index("s") * 2 + lax.axis_index("c")
        pltpu.sync_copy(idx_hbm.at[w], idx_v)
        handles = [
            pltpu.async_copy(x_hbm.at[idx_v.at[jj]], rows_v.at[jj], sem)
            for jj in range(NCH)
        ]
        for h in handles:
            h.wait()
        pltpu.sync_copy(rows_v, out_hbm.at[w])

    patches = _patch_gather(x_rows, idx3).reshape(N, 3 * PATCH * PATCH)
    w_patch = patch_W.reshape(D, 3 * PATCH * PATCH)
    mt = N // 4
    kv_emb = pl.pallas_call(
        _embed_kv_body,
        grid=(4,),
        in_specs=[
            pl.BlockSpec((mt, 3 * PATCH * PATCH), lambda i: (i, 0)),
            pl.BlockSpec(w_patch.shape, lambda i: (0, 0)),
            pl.BlockSpec((1, D), lambda i: (0, 0)),
            pl.BlockSpec((1, D), lambda i: (0, 0)),
            pl.BlockSpec((1, D), lambda i: (0, 0)),
            pl.BlockSpec((2 * D, D), lambda i: (0, 0)),
            pl.BlockSpec((1, 2 * D), lambda i: (0, 0)),
        ],
        out_specs=pl.BlockSpec((mt, 2 * D), lambda i: (i, 0)),
        out_shape=jax.ShapeDtypeStruct((N, 2 * D), jnp.float32),
    )(patches, w_patch, patch_b.reshape(1, D), ln1_w.reshape(1, D),
      ln1_b.reshape(1, D), attn_in_w[D:], attn_in_b[D:].reshape(1, 2 * D))
    kv = kv_emb.reshape(B, NP, 2 * D)

    xs1, xf, maskf = pl.pallas_call(
        _cls_block_body,
        out_shape=[
            jax.ShapeDtypeStruct((B, D), jnp.float32),
            jax.ShapeDtypeStruct((B, D), jnp.float32),
            jax.ShapeDtypeStruct((B, E), jnp.float32),
        ],
    )(cls.reshape(1, D), ln1_w.reshape(1, D), ln1_b.reshape(1, D),
      attn_in_w, attn_in_b.reshape(1, 3 * D), kv,
      attn_out_w, attn_out_b.reshape(1, D),
      ln2_w.reshape(1, D), ln2_b.reshape(1, D),
      router_W, router_b.reshape(1, E))

    b1r = exp_b1.reshape(E, NHC, 1, HCHUNK)
    b2r = exp_b2.reshape(E, 1, D)
    moe = pl.pallas_call(
        _moe_body,
        grid=(E, NHC),
        in_specs=[
            pl.BlockSpec((B, D), lambda e, h: (0, 0)),
            pl.BlockSpec((B, E), lambda e, h: (0, 0)),
            pl.BlockSpec((1, HCHUNK, D), lambda e, h: (e, h, 0)),
            pl.BlockSpec((1, 1, 1, HCHUNK), lambda e, h: (e, h, 0, 0)),
            pl.BlockSpec((1, D, HCHUNK), lambda e, h: (e, 0, h)),
            pl.BlockSpec((1, 1, D), lambda e, h: (e, 0, 0)),
        ],
        out_specs=pl.BlockSpec((B, D), lambda e, h: (0, 0)),
        out_shape=jax.ShapeDtypeStruct((B, D), jnp.float32),
    )(xf, maskf, exp_W1, b1r, exp_W2, b2r)

    out = pl.pallas_call(
        _head_body,
        out_shape=jax.ShapeDtypeStruct((B, NC), jnp.float32),
    )(xs1, moe, head_W, head_b.reshape(1, NC))
    return out


# speedup vs baseline: 3.6463x; 1.0110x over previous
import functools

import jax
import jax.numpy as jnp
from jax import lax
from jax.experimental import pallas as pl
from jax.experimental.pallas import tpu as pltpu
from jax.experimental.pallas import tpu_sc as plsc

D = 768
HEADS = 8
DH = D // HEADS
E = 8
DFF = 4 * D
HCHUNK = 768
NHC = DFF // HCHUNK
PATCH = 16

_DOT = functools.partial(
    jax.lax.dot_general,
    precision=jax.lax.Precision.HIGHEST,
    preferred_element_type=jnp.float32,
)
_DOTF = functools.partial(
    jax.lax.dot_general,
    precision=jax.lax.Precision.DEFAULT,
    preferred_element_type=jnp.float32,
)


def _mm_nt(a, b):
    return _DOT(a, b, dimension_numbers=(((a.ndim - 1,), (b.ndim - 1,)), ((), ())))


def _layernorm(x, w, b):
    m = jnp.mean(x, axis=-1, keepdims=True)
    v = jnp.mean((x - m) ** 2, axis=-1, keepdims=True)
    return (x - m) / jnp.sqrt(v + 1e-5) * w + b


def _embed_kv_body(p_ref, wp_ref, bp_ref, lw_ref, lb_ref, w_ref, b_ref, o_ref):
    emb = _mm_nt(p_ref[...], wp_ref[...]) + bp_ref[...]
    x2 = _layernorm(emb, lw_ref[...], lb_ref[...])
    o_ref[...] = _mm_nt(x2, w_ref[...]) + b_ref[...]


def _cls_block_body(clsr_ref, l1w_ref, l1b_ref, wqkv_ref, bqkv_ref, kv_ref,
                    wo_ref, bo_ref, l2w_ref, l2b_ref, rw_ref, rb_ref,
                    xs1_ref, xf_ref, mask_ref):
    nb = kv_ref.shape[0]
    clsr = clsr_ref[...]
    qkv_c = _mm_nt(_layernorm(clsr, l1w_ref[...], l1b_ref[...]),
                   wqkv_ref[...]) + bqkv_ref[...]
    qc = qkv_c[:, :D]
    kc = qkv_c[:, D:2 * D]
    vc = qkv_c[:, 2 * D:]
    hm = (jax.lax.broadcasted_iota(jnp.int32, (D, HEADS), 0) // DH
          == jax.lax.broadcasted_iota(jnp.int32, (D, HEADS), 1)).astype(jnp.float32)
    hmt = (jax.lax.broadcasted_iota(jnp.int32, (HEADS, D), 1) // DH
           == jax.lax.broadcasted_iota(jnp.int32, (HEADS, D), 0)).astype(jnp.float32)
    scale = 1.0 / jnp.sqrt(jnp.float32(DH))
    s_cls = _DOT(kc * qc, hm, (((1,), (0,)), ((), ()))) * scale
    npt = kv_ref.shape[1]
    kvf = kv_ref[...].reshape(nb * npt, 2 * D)
    kf = kvf[:, :D]
    vf = kvf[:, D:]
    sp = _DOT(kf * qc, hm, (((1,), (0,)), ((), ()))) * scale
    s3 = sp.reshape(nb, npt, HEADS)
    mx = jnp.maximum(jnp.max(s3, axis=1), s_cls)
    e3 = jnp.exp(s3 - mx[:, None, :])
    e_cls = jnp.exp(s_cls - mx)
    denom = jnp.sum(e3, axis=1) + e_cls
    attn_e = _DOT(e3.reshape(nb * npt, HEADS), hmt,
                  (((1,), (0,)), ((), ())))
    wv = jnp.sum((attn_e * vf).reshape(nb, npt, D), axis=1)
    wv = wv + _DOT(e_cls, hmt, (((1,), (0,)), ((), ()))) * vc
    o = wv / _DOT(denom, hmt, (((1,), (0,)), ((), ())))
    xs1 = clsr + _mm_nt(o, wo_ref[...]) + bo_ref[...]
    xf = _layernorm(xs1, l2w_ref[...], l2b_ref[...])
    logits = _mm_nt(xf, rw_ref[...]) + rb_ref[...]
    lane = jax.lax.broadcasted_iota(jnp.int32, (nb, E), 1)
    cnt = jnp.zeros((nb, E), jnp.float32)
    for ep in range(E):
        le = logits[:, ep:ep + 1]
        gt = (le > logits).astype(jnp.float32)
        eq = jnp.logical_and(le == logits, ep < lane).astype(jnp.float32)
        cnt = cnt + gt + eq
    mask_ref[...] = (cnt < 2.0).astype(jnp.float32)
    xs1_ref[...] = xs1
    xf_ref[...] = xf


def _moe_head_body(xf_ref, m_ref, w1_ref, b1_ref, w2_ref, b2_ref,
                   xs1_ref, hw_ref, hb_ref, o_ref, acc_ref):
    e = pl.program_id(0)
    hc = pl.program_id(1)

    @pl.when(jnp.logical_and(e == 0, hc == 0))
    def _():
        acc_ref[...] = jnp.zeros_like(acc_ref)

    xf = xf_ref[...]
    h = _DOTF(xf, w1_ref[0], (((1,), (1,)), ((), ()))) + b1_ref[0, 0]
    g = h * 0.5 * (1.0 + jax.lax.erf(h * (2.0 ** -0.5)))
    contrib = _DOTF(g, w2_ref[0], (((1,), (1,)), ((), ())))
    onehot = (jax.lax.broadcasted_iota(jnp.int32, (E, 1), 0) == e).astype(jnp.float32)
    mcol = _DOT(m_ref[...], onehot, (((1,), (0,)), ((), ())))

    @pl.when(hc == 0)
    def _():
        acc_ref[...] += mcol * b2_ref[0]

    acc_ref[...] += mcol * contrib

    @pl.when(jnp.logical_and(e == E - 1, hc == NHC - 1))
    def _():
        y = xs1_ref[...] + acc_ref[...]
        o_ref[...] = _mm_nt(y, hw_ref[...]) + hb_ref[...]


def kernel(x, patch_W, patch_b, cls, ln1_w, ln1_b, attn_in_w, attn_in_b,
           attn_out_w, attn_out_b, ln2_w, ln2_b, router_W, router_b,
           exp_W1, exp_b1, exp_W2, exp_b2, head_W, head_b):
    B = x.shape[0]
    Hp = x.shape[2] // PATCH
    Wp = x.shape[3] // PATCH
    NP = Hp * Wp
    N = B * NP
    NC = head_W.shape[0]

    NSEG = 3 * PATCH
    NROWS = N * NSEG
    NW = 32
    per_w = NROWS // NW
    NCHB = 112
    NCH = per_w // NCHB
    j = jnp.arange(NROWS, dtype=jnp.int32)
    t, s = j // NSEG, j % NSEG
    b_i, r_i = t // NP, t % NP
    hp_i, wp_i = r_i // Wp, r_i % Wp
    c_i, py_i = s // PATCH, s % PATCH
    src = ((b_i * 3 + c_i) * (Hp * PATCH) + hp_i * PATCH + py_i) * Wp + wp_i
    idx3 = src.reshape(NW, NCH, NCHB)
    x_rows = x.reshape(NROWS, PATCH)

    mesh = plsc.VectorSubcoreMesh(core_axis_name="c", subcore_axis_name="s")

    @functools.partial(
        pl.kernel,
        mesh=mesh,
        compiler_params=pltpu.CompilerParams(use_tc_tiling_on_sc=False),
        out_type=jax.ShapeDtypeStruct((NW, NCH, NCHB, PATCH), jnp.float32),
        scratch_types=[
            pltpu.VMEM((NCH, NCHB), jnp.int32),
            pltpu.VMEM((NCH, NCHB, PATCH), jnp.float32),
            pltpu.SemaphoreType.DMA,
        ],
    )
    def _patch_gather(x_hbm, idx_hbm, out_hbm, idx_v, rows_v, sem):
        w = lax.axis_index("s") * 2 + lax.axis_index("c")
        pltpu.sync_copy(idx_hbm.at[w], idx_v)
        handles = [
            pltpu.async_copy(x_hbm.at[idx_v.at[jj]], rows_v.at[jj], sem)
            for jj in range(NCH)
        ]
        for h in handles:
            h.wait()
        pltpu.sync_copy(rows_v, out_hbm.at[w])

    patches = _patch_gather(x_rows, idx3).reshape(N, 3 * PATCH * PATCH)
    w_patch = patch_W.reshape(D, 3 * PATCH * PATCH)
    mt = N // 4
    kv_emb = pl.pallas_call(
        _embed_kv_body,
        grid=(4,),
        in_specs=[
            pl.BlockSpec((mt, 3 * PATCH * PATCH), lambda i: (i, 0)),
            pl.BlockSpec(w_patch.shape, lambda i: (0, 0)),
            pl.BlockSpec((1, D), lambda i: (0, 0)),
            pl.BlockSpec((1, D), lambda i: (0, 0)),
            pl.BlockSpec((1, D), lambda i: (0, 0)),
            pl.BlockSpec((2 * D, D), lambda i: (0, 0)),
            pl.BlockSpec((1, 2 * D), lambda i: (0, 0)),
        ],
        out_specs=pl.BlockSpec((mt, 2 * D), lambda i: (i, 0)),
        out_shape=jax.ShapeDtypeStruct((N, 2 * D), jnp.float32),
    )(patches, w_patch, patch_b.reshape(1, D), ln1_w.reshape(1, D),
      ln1_b.reshape(1, D), attn_in_w[D:], attn_in_b[D:].reshape(1, 2 * D))
    kv = kv_emb.reshape(B, NP, 2 * D)

    xs1, xf, maskf = pl.pallas_call(
        _cls_block_body,
        out_shape=[
            jax.ShapeDtypeStruct((B, D), jnp.float32),
            jax.ShapeDtypeStruct((B, D), jnp.float32),
            jax.ShapeDtypeStruct((B, E), jnp.float32),
        ],
    )(cls.reshape(1, D), ln1_w.reshape(1, D), ln1_b.reshape(1, D),
      attn_in_w, attn_in_b.reshape(1, 3 * D), kv,
      attn_out_w, attn_out_b.reshape(1, D),
      ln2_w.reshape(1, D), ln2_b.reshape(1, D),
      router_W, router_b.reshape(1, E))

    b1r = exp_b1.reshape(E, NHC, 1, HCHUNK)
    b2r = exp_b2.reshape(E, 1, D)
    out = pl.pallas_call(
        _moe_head_body,
        grid=(E, NHC),
        in_specs=[
            pl.BlockSpec((B, D), lambda e, h: (0, 0)),
            pl.BlockSpec((B, E), lambda e, h: (0, 0)),
            pl.BlockSpec((1, HCHUNK, D), lambda e, h: (e, h, 0)),
            pl.BlockSpec((1, 1, 1, HCHUNK), lambda e, h: (e, h, 0, 0)),
            pl.BlockSpec((1, D, HCHUNK), lambda e, h: (e, 0, h)),
            pl.BlockSpec((1, 1, D), lambda e, h: (e, 0, 0)),
            pl.BlockSpec((B, D), lambda e, h: (0, 0)),
            pl.BlockSpec((NC, D), lambda e, h: (0, 0)),
            pl.BlockSpec((1, NC), lambda e, h: (0, 0)),
        ],
        out_specs=pl.BlockSpec((B, NC), lambda e, h: (0, 0)),
        out_shape=jax.ShapeDtypeStruct((B, NC), jnp.float32),
        scratch_shapes=[pltpu.VMEM((B, D), jnp.float32)],
    )(xf, maskf, exp_W1, b1r, exp_W2, b2r, xs1, head_W,
      head_b.reshape(1, NC))
    return out


# speedup vs baseline: 4.8486x; 1.3297x over previous
import functools

import jax
import jax.numpy as jnp
from jax import lax
from jax.experimental import pallas as pl
from jax.experimental.pallas import tpu as pltpu
from jax.experimental.pallas import tpu_sc as plsc

D = 768
HEADS = 8
DH = D // HEADS
E = 8
DFF = 4 * D
HCHUNK = 768
NHC = DFF // HCHUNK
PATCH = 16

_DOTF = functools.partial(
    jax.lax.dot_general,
    precision=jax.lax.Precision.DEFAULT,
    preferred_element_type=jnp.float32,
)


def _b16(x):
    return x.astype(jnp.bfloat16).astype(jnp.float32)


def _layernorm(x, w, b):
    m = jnp.mean(x, axis=-1, keepdims=True)
    v = jnp.mean((x - m) ** 2, axis=-1, keepdims=True)
    return (x - m) / jnp.sqrt(v + 1e-5) * w + b


def _embed_kv_body(p_ref, wp_ref, bp_ref, lw_ref, lb_ref, w_ref, b_ref, o_ref):
    emb = _DOTF(p_ref[...], wp_ref[...],
                (((1,), (1,)), ((), ()))) + bp_ref[...]
    x2 = _layernorm(emb, lw_ref[...], lb_ref[...])
    o_ref[...] = _DOTF(x2, w_ref[...], (((1,), (1,)), ((), ()))) + b_ref[...]


def _cls_block_body(clsr_ref, l1w_ref, l1b_ref, wqkv_ref, bqkv_ref, kv_ref,
                    wo_ref, bo_ref, l2w_ref, l2b_ref, rw_ref, rb_ref,
                    xs1_ref, xf_ref, mask_ref):
    nb = kv_ref.shape[0]
    clsr = clsr_ref[...]
    qkv_c = _DOTF(_layernorm(clsr, l1w_ref[...], l1b_ref[...]),
                  wqkv_ref[...],
                  (((1,), (1,)), ((), ()))) + bqkv_ref[...]
    qc = qkv_c[:, :D]
    kc = qkv_c[:, D:2 * D]
    vc = qkv_c[:, 2 * D:]
    hm = (jax.lax.broadcasted_iota(jnp.int32, (D, HEADS), 0) // DH
          == jax.lax.broadcasted_iota(jnp.int32, (D, HEADS), 1)).astype(jnp.bfloat16)
    hmt = (jax.lax.broadcasted_iota(jnp.int32, (HEADS, D), 1) // DH
           == jax.lax.broadcasted_iota(jnp.int32, (HEADS, D), 0)).astype(jnp.bfloat16)
    rootdh = jnp.sqrt(jnp.float32(DH))
    npt = kv_ref.shape[1]
    kvf = kv_ref[...].reshape(nb * npt, 2 * D)

    def _head_sums(prod):
        hi = prod.astype(jnp.bfloat16)
        lo = (prod - hi.astype(jnp.float32)).astype(jnp.bfloat16)
        return (_DOTF(hi, hm, (((1,), (0,)), ((), ())))
                + _DOTF(lo, hm, (((1,), (0,)), ((), ()))))

    q16 = _b16(qc)
    s_cls = _head_sums(_b16(kc) * q16) / rootdh
    sp = _head_sums(_b16(kvf[:, :D]) * q16) / rootdh
    s3 = sp.reshape(nb, npt, HEADS)
    mx = jnp.maximum(jnp.max(s3, axis=1), s_cls)
    e3 = jnp.exp(s3 - mx[:, None, :])
    e_cls = jnp.exp(s_cls - mx)
    denom = jnp.sum(e3, axis=1) + e_cls
    attn3 = e3 / denom[:, None, :]
    attn_cls = e_cls / denom
    attn_e = _DOTF(attn3.reshape(nb * npt, HEADS).astype(jnp.bfloat16),
                   hmt, (((1,), (0,)), ((), ())))
    wv = jnp.sum((attn_e * _b16(kvf[:, D:])).reshape(nb, npt, D), axis=1)
    ac_e = _DOTF(attn_cls.astype(jnp.bfloat16), hmt, (((1,), (0,)), ((), ())))
    o = wv + ac_e * _b16(vc)
    xs1 = clsr + _DOTF(o, wo_ref[...], (((1,), (1,)), ((), ()))) + bo_ref[...]
    xf = _layernorm(xs1, l2w_ref[...], l2b_ref[...])
    logits = _DOTF(xf, rw_ref[...], (((1,), (1,)), ((), ()))) + rb_ref[...]
    lane = jax.lax.broadcasted_iota(jnp.int32, (nb, E), 1)
    cnt = jnp.zeros((nb, E), jnp.float32)
    for ep in range(E):
        le = logits[:, ep:ep + 1]
        gt = (le > logits).astype(jnp.float32)
        eq = jnp.logical_and(le == logits, ep < lane).astype(jnp.float32)
        cnt = cnt + gt + eq
    mask_ref[...] = (cnt < 2.0).astype(jnp.float32)
    xs1_ref[...] = xs1
    xf_ref[...] = xf


def _moe_head_body(xf_ref, m_ref, w1_ref, b1_ref, w2_ref, b2_ref,
                   xs1_ref, hw_ref, hb_ref, o_ref, acc_ref):
    e = pl.program_id(0)
    hc = pl.program_id(1)

    @pl.when(jnp.logical_and(e == 0, hc == 0))
    def _():
        acc_ref[...] = jnp.zeros_like(acc_ref)

    xf = xf_ref[...]
    h = _DOTF(xf, w1_ref[0], (((1,), (1,)), ((), ()))) + b1_ref[0, 0]
    g = h * 0.5 * (1.0 + jax.lax.erf(h * (2.0 ** -0.5)))
    contrib = _DOTF(g, w2_ref[0], (((1,), (1,)), ((), ())))
    onehot = (jax.lax.broadcasted_iota(jnp.int32, (E, 1), 0) == e).astype(jnp.float32)
    mcol = _DOTF(m_ref[...], onehot, (((1,), (0,)), ((), ())))

    @pl.when(hc == 0)
    def _():
        acc_ref[...] += mcol * b2_ref[0]

    acc_ref[...] += mcol * contrib

    @pl.when(jnp.logical_and(e == E - 1, hc == NHC - 1))
    def _():
        y = xs1_ref[...] + acc_ref[...]
        o_ref[...] = _DOTF(y, hw_ref[...], (((1,), (1,)), ((), ()))) + hb_ref[...]


def kernel(x, patch_W, patch_b, cls, ln1_w, ln1_b, attn_in_w, attn_in_b,
           attn_out_w, attn_out_b, ln2_w, ln2_b, router_W, router_b,
           exp_W1, exp_b1, exp_W2, exp_b2, head_W, head_b):
    B = x.shape[0]
    Hp = x.shape[2] // PATCH
    Wp = x.shape[3] // PATCH
    NP = Hp * Wp
    N = B * NP
    NC = head_W.shape[0]

    NSEG = 3 * PATCH
    NROWS = N * NSEG
    NW = 32
    per_w = NROWS // NW
    NCHB = 112
    NCH = per_w // NCHB
    j = jnp.arange(NROWS, dtype=jnp.int32)
    t, s = j // NSEG, j % NSEG
    b_i, r_i = t // NP, t % NP
    hp_i, wp_i = r_i // Wp, r_i % Wp
    c_i, py_i = s // PATCH, s % PATCH
    src = ((b_i * 3 + c_i) * (Hp * PATCH) + hp_i * PATCH + py_i) * Wp + wp_i
    idx3 = src.reshape(NW, NCH, NCHB)
    x_rows = x.reshape(NROWS, PATCH)

    mesh = plsc.VectorSubcoreMesh(core_axis_name="c", subcore_axis_name="s")

    @functools.partial(
        pl.kernel,
        mesh=mesh,
        compiler_params=pltpu.CompilerParams(use_tc_tiling_on_sc=False),
        out_type=jax.ShapeDtypeStruct((NW, NCH, NCHB, PATCH), jnp.float32),
        scratch_types=[
            pltpu.VMEM((NCH, NCHB), jnp.int32),
            pltpu.VMEM((NCH, NCHB, PATCH), jnp.float32),
            pltpu.SemaphoreType.DMA,
        ],
    )
    def _patch_gather(x_hbm, idx_hbm, out_hbm, idx_v, rows_v, sem):
        w = lax.axis_index("s") * 2 + lax.axis_index("c")
        pltpu.sync_copy(idx_hbm.at[w], idx_v)
        handles = [
            pltpu.async_copy(x_hbm.at[idx_v.at[jj]], rows_v.at[jj], sem)
            for jj in range(NCH)
        ]
        for h in handles:
            h.wait()
        pltpu.sync_copy(rows_v, out_hbm.at[w])

    patches = _patch_gather(x_rows, idx3).reshape(N, 3 * PATCH * PATCH)
    w_patch = patch_W.reshape(D, 3 * PATCH * PATCH)
    mt = N // 4
    kv_emb = pl.pallas_call(
        _embed_kv_body,
        grid=(4,),
        in_specs=[
            pl.BlockSpec((mt, 3 * PATCH * PATCH), lambda i: (i, 0)),
            pl.BlockSpec(w_patch.shape, lambda i: (0, 0)),
            pl.BlockSpec((1, D), lambda i: (0, 0)),
            pl.BlockSpec((1, D), lambda i: (0, 0)),
            pl.BlockSpec((1, D), lambda i: (0, 0)),
            pl.BlockSpec((2 * D, D), lambda i: (0, 0)),
            pl.BlockSpec((1, 2 * D), lambda i: (0, 0)),
        ],
        out_specs=pl.BlockSpec((mt, 2 * D), lambda i: (i, 0)),
        out_shape=jax.ShapeDtypeStruct((N, 2 * D), jnp.float32),
    )(patches, w_patch, patch_b.reshape(1, D), ln1_w.reshape(1, D),
      ln1_b.reshape(1, D), attn_in_w[D:], attn_in_b[D:].reshape(1, 2 * D))
    kv = kv_emb.reshape(B, NP, 2 * D)

    xs1, xf, maskf = pl.pallas_call(
        _cls_block_body,
        out_shape=[
            jax.ShapeDtypeStruct((B, D), jnp.float32),
            jax.ShapeDtypeStruct((B, D), jnp.float32),
            jax.ShapeDtypeStruct((B, E), jnp.float32),
        ],
    )(cls.reshape(1, D), ln1_w.reshape(1, D), ln1_b.reshape(1, D),
      attn_in_w, attn_in_b.reshape(1, 3 * D), kv,
      attn_out_w, attn_out_b.reshape(1, D),
      ln2_w.reshape(1, D), ln2_b.reshape(1, D),
      router_W, router_b.reshape(1, E))

    b1r = exp_b1.reshape(E, NHC, 1, HCHUNK)
    b2r = exp_b2.reshape(E, 1, D)
    out = pl.pallas_call(
        _moe_head_body,
        grid=(E, NHC),
        in_specs=[
            pl.BlockSpec((B, D), lambda e, h: (0, 0)),
            pl.BlockSpec((B, E), lambda e, h: (0, 0)),
            pl.BlockSpec((1, HCHUNK, D), lambda e, h: (e, h, 0)),
            pl.BlockSpec((1, 1, 1, HCHUNK), lambda e, h: (e, h, 0, 0)),
            pl.BlockSpec((1, D, HCHUNK), lambda e, h: (e, 0, h)),
            pl.BlockSpec((1, 1, D), lambda e, h: (e, 0, 0)),
            pl.BlockSpec((B, D), lambda e, h: (0, 0)),
            pl.BlockSpec((NC, D), lambda e, h: (0, 0)),
            pl.BlockSpec((1, NC), lambda e, h: (0, 0)),
        ],
        out_specs=pl.BlockSpec((B, NC), lambda e, h: (0, 0)),
        out_shape=jax.ShapeDtypeStruct((B, NC), jnp.float32),
        scratch_shapes=[pltpu.VMEM((B, D), jnp.float32)],
    )(xf, maskf, exp_W1, b1r, exp_W2, b2r, xs1, head_W,
      head_b.reshape(1, NC))
    return out


# speedup vs baseline: 4.8787x; 1.0062x over previous
import functools

import jax
import jax.numpy as jnp
from jax import lax
from jax.experimental import pallas as pl
from jax.experimental.pallas import tpu as pltpu
from jax.experimental.pallas import tpu_sc as plsc

D = 768
HEADS = 8
DH = D // HEADS
E = 8
DFF = 4 * D
HCHUNK = 768
NHC = DFF // HCHUNK
PATCH = 16

_DOTF = functools.partial(
    jax.lax.dot_general,
    precision=jax.lax.Precision.DEFAULT,
    preferred_element_type=jnp.float32,
)


def _b16(x):
    return x.astype(jnp.bfloat16).astype(jnp.float32)


def _layernorm(x, w, b):
    m = jnp.mean(x, axis=-1, keepdims=True)
    v = jnp.mean((x - m) ** 2, axis=-1, keepdims=True)
    return (x - m) / jnp.sqrt(v + 1e-5) * w + b


def _embed_kv_body(p_ref, wp_ref, bp_ref, lw_ref, lb_ref, w_ref, b_ref, o_ref):
    emb = _DOTF(p_ref[...], wp_ref[...],
                (((1,), (1,)), ((), ()))) + bp_ref[...]
    x2 = _layernorm(emb, lw_ref[...], lb_ref[...])
    o_ref[...] = _DOTF(x2, w_ref[...], (((1,), (1,)), ((), ()))) + b_ref[...]


def _cls_block_body(clsr_ref, l1w_ref, l1b_ref, wqkv_ref, bqkv_ref, kv_ref,
                    wo_ref, bo_ref, l2w_ref, l2b_ref, rw_ref, rb_ref,
                    xs1_ref, xf_ref, mask_ref):
    nb = kv_ref.shape[0]
    clsr = clsr_ref[...]
    qkv_c = _DOTF(_layernorm(clsr, l1w_ref[...], l1b_ref[...]),
                  wqkv_ref[...],
                  (((1,), (1,)), ((), ()))) + bqkv_ref[...]
    qc = qkv_c[:, :D]
    kc = qkv_c[:, D:2 * D]
    vc = qkv_c[:, 2 * D:]
    hm = (jax.lax.broadcasted_iota(jnp.int32, (D, HEADS), 0) // DH
          == jax.lax.broadcasted_iota(jnp.int32, (D, HEADS), 1)).astype(jnp.bfloat16)
    hmt = (jax.lax.broadcasted_iota(jnp.int32, (HEADS, D), 1) // DH
           == jax.lax.broadcasted_iota(jnp.int32, (HEADS, D), 0)).astype(jnp.bfloat16)
    rootdh = jnp.sqrt(jnp.float32(DH))
    npt = kv_ref.shape[1]
    kvf = kv_ref[...].reshape(nb * npt, 2 * D)

    def _head_sums(prod):
        hi = prod.astype(jnp.bfloat16)
        lo = (prod - hi.astype(jnp.float32)).astype(jnp.bfloat16)
        return (_DOTF(hi, hm, (((1,), (0,)), ((), ())))
                + _DOTF(lo, hm, (((1,), (0,)), ((), ()))))

    q16 = _b16(qc)
    s_cls = _head_sums(_b16(kc) * q16) / rootdh
    sp = _head_sums(_b16(kvf[:, :D]) * q16) / rootdh
    s3 = sp.reshape(nb, npt, HEADS)
    mx = jnp.maximum(jnp.max(s3, axis=1), s_cls)
    e3 = jnp.exp(s3 - mx[:, None, :])
    e_cls = jnp.exp(s_cls - mx)
    denom = jnp.sum(e3, axis=1) + e_cls
    attn3 = e3 / denom[:, None, :]
    attn_cls = e_cls / denom
    attn_e = _DOTF(attn3.reshape(nb * npt, HEADS).astype(jnp.bfloat16),
                   hmt, (((1,), (0,)), ((), ())))
    wv = jnp.sum((attn_e * _b16(kvf[:, D:])).reshape(nb, npt, D), axis=1)
    ac_e = _DOTF(attn_cls.astype(jnp.bfloat16), hmt, (((1,), (0,)), ((), ())))
    o = wv + ac_e * _b16(vc)
    xs1 = clsr + _DOTF(o, wo_ref[...], (((1,), (1,)), ((), ()))) + bo_ref[...]
    xf = _layernorm(xs1, l2w_ref[...], l2b_ref[...])
    logits = _DOTF(xf, rw_ref[...], (((1,), (1,)), ((), ()))) + rb_ref[...]
    lane = jax.lax.broadcasted_iota(jnp.int32, (nb, E), 1)
    cnt = jnp.zeros((nb, E), jnp.float32)
    for ep in range(E):
        le = logits[:, ep:ep + 1]
        gt = (le > logits).astype(jnp.float32)
        eq = jnp.logical_and(le == logits, ep < lane).astype(jnp.float32)
        cnt = cnt + gt + eq
    mask_ref[...] = (cnt < 2.0).astype(jnp.float32)
    xs1_ref[...] = xs1
    xf_ref[...] = xf


def _moe_head_body(xf_ref, m_ref, w1_ref, b1_ref, w2_ref, b2_ref,
                   xs1_ref, hw_ref, hb_ref, o_ref, acc_ref):
    e = pl.program_id(0)
    hc = pl.program_id(1)

    @pl.when(jnp.logical_and(e == 0, hc == 0))
    def _():
        acc_ref[...] = jnp.zeros_like(acc_ref)

    xf = xf_ref[...]
    h = _DOTF(xf, w1_ref[0], (((1,), (1,)), ((), ()))) + b1_ref[0, 0]
    g = h * 0.5 * (1.0 + jax.lax.erf(h * (2.0 ** -0.5)))
    contrib = _DOTF(g, w2_ref[0], (((1,), (1,)), ((), ())))
    onehot = (jax.lax.broadcasted_iota(jnp.int32, (E, 1), 0) == e).astype(jnp.float32)
    mcol = _DOTF(m_ref[...], onehot, (((1,), (0,)), ((), ())))

    @pl.when(hc == 0)
    def _():
        acc_ref[...] += mcol * b2_ref[0]

    acc_ref[...] += mcol * contrib

    @pl.when(jnp.logical_and(e == E - 1, hc == NHC - 1))
    def _():
        y = xs1_ref[...] + acc_ref[...]
        o_ref[...] = _DOTF(y, hw_ref[...], (((1,), (1,)), ((), ()))) + hb_ref[...]


def kernel(x, patch_W, patch_b, cls, ln1_w, ln1_b, attn_in_w, attn_in_b,
           attn_out_w, attn_out_b, ln2_w, ln2_b, router_W, router_b,
           exp_W1, exp_b1, exp_W2, exp_b2, head_W, head_b):
    B = x.shape[0]
    Hp = x.shape[2] // PATCH
    Wp = x.shape[3] // PATCH
    NP = Hp * Wp
    N = B * NP
    NC = head_W.shape[0]

    NSEG = 3 * PATCH
    NROWS = N * NSEG
    NW = 32
    per_w = NROWS // NW
    NCHB = 112
    NCH = per_w // NCHB
    j = jnp.arange(NROWS, dtype=jnp.int32)
    t, s = j // NSEG, j % NSEG
    b_i, r_i = t // NP, t % NP
    hp_i, wp_i = r_i // Wp, r_i % Wp
    c_i, py_i = s // PATCH, s % PATCH
    src = ((b_i * 3 + c_i) * (Hp * PATCH) + hp_i * PATCH + py_i) * Wp + wp_i
    idx3 = src.reshape(NW, NCH, NCHB)
    x_rows = x.reshape(NROWS, PATCH)

    mesh = plsc.VectorSubcoreMesh(core_axis_name="c", subcore_axis_name="s")

    @functools.partial(
        pl.kernel,
        mesh=mesh,
        compiler_params=pltpu.CompilerParams(use_tc_tiling_on_sc=False),
        out_type=jax.ShapeDtypeStruct((NW, NCH, NCHB, PATCH), jnp.float32),
        scratch_types=[
            pltpu.VMEM((NCH, NCHB), jnp.int32),
            pltpu.VMEM((NCH, NCHB, PATCH), jnp.float32),
            pltpu.SemaphoreType.DMA,
        ],
    )
    def _patch_gather(x_hbm, idx_hbm, out_hbm, idx_v, rows_v, sem):
        w = lax.axis_index("s") * 2 + lax.axis_index("c")
        pltpu.sync_copy(idx_hbm.at[w], idx_v)
        handles = [
            pltpu.async_copy(x_hbm.at[idx_v.at[jj]], rows_v.at[jj], sem)
            for jj in range(NCH)
        ]
        for jj, h in enumerate(handles):
            h.wait()
            pltpu.sync_copy(rows_v.at[jj], out_hbm.at[w, jj])

    patches = _patch_gather(x_rows, idx3).reshape(N, 3 * PATCH * PATCH)
    w_patch = patch_W.reshape(D, 3 * PATCH * PATCH)
    mt = N // 4
    kv_emb = pl.pallas_call(
        _embed_kv_body,
        grid=(4,),
        in_specs=[
            pl.BlockSpec((mt, 3 * PATCH * PATCH), lambda i: (i, 0)),
            pl.BlockSpec(w_patch.shape, lambda i: (0, 0)),
            pl.BlockSpec((1, D), lambda i: (0, 0)),
            pl.BlockSpec((1, D), lambda i: (0, 0)),
            pl.BlockSpec((1, D), lambda i: (0, 0)),
            pl.BlockSpec((2 * D, D), lambda i: (0, 0)),
            pl.BlockSpec((1, 2 * D), lambda i: (0, 0)),
        ],
        out_specs=pl.BlockSpec((mt, 2 * D), lambda i: (i, 0)),
        out_shape=jax.ShapeDtypeStruct((N, 2 * D), jnp.float32),
    )(patches, w_patch, patch_b.reshape(1, D), ln1_w.reshape(1, D),
      ln1_b.reshape(1, D), attn_in_w[D:], attn_in_b[D:].reshape(1, 2 * D))
    kv = kv_emb.reshape(B, NP, 2 * D)

    xs1, xf, maskf = pl.pallas_call(
        _cls_block_body,
        out_shape=[
            jax.ShapeDtypeStruct((B, D), jnp.float32),
            jax.ShapeDtypeStruct((B, D), jnp.float32),
            jax.ShapeDtypeStruct((B, E), jnp.float32),
        ],
    )(cls.reshape(1, D), ln1_w.reshape(1, D), ln1_b.reshape(1, D),
      attn_in_w, attn_in_b.reshape(1, 3 * D), kv,
      attn_out_w, attn_out_b.reshape(1, D),
      ln2_w.reshape(1, D), ln2_b.reshape(1, D),
      router_W, router_b.reshape(1, E))

    b1r = exp_b1.reshape(E, NHC, 1, HCHUNK)
    b2r = exp_b2.reshape(E, 1, D)
    out = pl.pallas_call(
        _moe_head_body,
        grid=(E, NHC),
        in_specs=[
            pl.BlockSpec((B, D), lambda e, h: (0, 0)),
            pl.BlockSpec((B, E), lambda e, h: (0, 0)),
            pl.BlockSpec((1, HCHUNK, D), lambda e, h: (e, h, 0)),
            pl.BlockSpec((1, 1, 1, HCHUNK), lambda e, h: (e, h, 0, 0)),
            pl.BlockSpec((1, D, HCHUNK), lambda e, h: (e, 0, h)),
            pl.BlockSpec((1, 1, D), lambda e, h: (e, 0, 0)),
            pl.BlockSpec((B, D), lambda e, h: (0, 0)),
            pl.BlockSpec((NC, D), lambda e, h: (0, 0)),
            pl.BlockSpec((1, NC), lambda e, h: (0, 0)),
        ],
        out_specs=pl.BlockSpec((B, NC), lambda e, h: (0, 0)),
        out_shape=jax.ShapeDtypeStruct((B, NC), jnp.float32),
        scratch_shapes=[pltpu.VMEM((B, D), jnp.float32)],
    )(xf, maskf, exp_W1, b1r, exp_W2, b2r, xs1, head_W,
      head_b.reshape(1, NC))
    return out
